# baseline (speedup 1.0000x reference)
import jax
import jax.numpy as jnp
import numpy as np
from jax.experimental import pallas as pl
from jax.experimental.pallas import tpu as pltpu

_N_STATES = 16
_N_SYMBOLS = 8
_SEQ_LEN = 200

_DFA = {0: {'0': 1, '1': 2, 'and(2,3)': 3}, 1: {'2': 3, 'not(0)': 0, '4': 5}, 2: {'or(1,5)': 4, '3': 2}, 3: {'5': 6, 'T': 0}, 4: {'6': 7, 'and(0,not(1))': 8}, 5: {'7': 9, '2': 5}, 6: {'or(and(0,1),2)': 10, '4': 6}, 7: {'1': 11, 'not(6)': 7}, 8: {'3': 12, '0': 8}, 9: {'5': 13, 'or(2,3)': 9}, 10: {'and(4,5)': 14, '6': 10}, 11: {'7': 15, '1': 11}, 12: {'0': 0, 'not(7)': 12}, 13: {'2': 1, '6': 13}, 14: {'or(0,not(4))': 2, '3': 14}, 15: {'T': 3}}

_TRANS = [(s, g, d) for s in sorted(_DFA.keys()) for g, d in _DFA[s].items()]

_OUT_EDGES = {}
for _t, (_s, _g, _d) in enumerate(_TRANS):
    _OUT_EDGES.setdefault(_s, []).append((_t, _d))
_PATHS = {}
for _e1, (_s, _g, _k) in enumerate(_TRANS):
    for (_e2, _d) in _OUT_EDGES.get(_k, []):
        _PATHS.setdefault((_d, _s), []).append((_e1, _e2))


def _divide_args(guard):
    args = guard.split(',')
    out = []
    i = 0
    while i < len(args):
        a = args[i]
        while a.count('(') != a.count(')'):
            i += 1
            a = a + ',' + args[i]
        out.append(a)
        i += 1
    return out


def _eval_guard(guard, cols):
    if guard[0] == 'a':
        v = 1.0
        for a in _divide_args(guard[4:-1]):
            v = v * _eval_guard(a, cols)
        return v
    elif guard[0] == 'o':
        v = 0.0
        for a in _divide_args(guard[3:-1]):
            e = _eval_guard(a, cols)
            v = v + e - v * e
        return v
    elif guard[0] == 'n':
        return 1.0 - _eval_guard(guard[4:-1], cols)
    elif guard[0] == 'T':
        return jnp.ones_like(cols[0])
    else:
        return cols[int(guard)]


def _build_pair_mats(p_pairs):
    cols_e = [p_pairs[:, k:k + 1] for k in range(_N_SYMBOLS)]
    cols_o = [p_pairs[:, _N_SYMBOLS + k:_N_SYMBOLS + k + 1]
              for k in range(_N_SYMBOLS)]
    ge = [_eval_guard(g, cols_e) for (_, g, _) in _TRANS]
    go = [_eval_guard(g, cols_o) for (_, g, _) in _TRANS]
    zero = jnp.zeros_like(cols_e[0])

    rows = []
    for d in range(_N_STATES):
        row = []
        for s in range(_N_STATES):
            acc = None
            for (e1, e2) in _PATHS.get((d, s), ()):
                term = go[e2] * ge[e1]
                acc = term if acc is None else acc + term
            row.append(zero if acc is None else acc)
        rows.append(jnp.concatenate(row, axis=1)[:, None, :])
    return jnp.concatenate(rows, axis=1)


def _dot(x, y):
    return jax.lax.dot_general(
        x, y, (((1,), (0,)), ((), ())),
        precision=jax.lax.Precision.HIGHEST,
        preferred_element_type=jnp.float32)


def _fuzzy_kernel(p_ref, out_ref, b_ref, c_ref):
    b_ref[:, :, :] = _build_pair_mats(p_ref[:, :])

    for i in range(50):
        c_ref[i] = _dot(b_ref[2 * i + 1], b_ref[2 * i])
    mats = [_dot(c_ref[2 * i + 1], c_ref[2 * i]) for i in range(25)]
    while len(mats) > 1:
        nxt = [_dot(mats[2 * i + 1], mats[2 * i]) for i in range(len(mats) // 2)]
        if len(mats) % 2:
            nxt.append(mats[-1])
        mats = nxt

    out_ref[:, :] = mats[0]


def kernel(symbols_prob):
    p_pairs = symbols_prob.reshape(_SEQ_LEN // 2, 2 * _N_SYMBOLS)
    out = pl.pallas_call(
        _fuzzy_kernel,
        out_shape=jax.ShapeDtypeStruct((_N_STATES, _N_STATES), symbols_prob.dtype),
        compiler_params=pltpu.CompilerParams(
            skip_device_barrier=True,
        ),
        scratch_shapes=[
            pltpu.VMEM((_SEQ_LEN // 2, _N_STATES, _N_STATES), symbols_prob.dtype),
            pltpu.VMEM((_SEQ_LEN // 4, _N_STATES, _N_STATES), symbols_prob.dtype),
        ],
    )(p_pairs)
    return out[:, 0]


# speedup vs baseline: 1.2490x; 1.2490x over previous
import jax
import jax.numpy as jnp
import numpy as np
from jax.experimental import pallas as pl
from jax.experimental.pallas import tpu as pltpu

_N_STATES = 16
_N_SYMBOLS = 8
_SEQ_LEN = 200

_DFA = {0: {'0': 1, '1': 2, 'and(2,3)': 3}, 1: {'2': 3, 'not(0)': 0, '4': 5}, 2: {'or(1,5)': 4, '3': 2}, 3: {'5': 6, 'T': 0}, 4: {'6': 7, 'and(0,not(1))': 8}, 5: {'7': 9, '2': 5}, 6: {'or(and(0,1),2)': 10, '4': 6}, 7: {'1': 11, 'not(6)': 7}, 8: {'3': 12, '0': 8}, 9: {'5': 13, 'or(2,3)': 9}, 10: {'and(4,5)': 14, '6': 10}, 11: {'7': 15, '1': 11}, 12: {'0': 0, 'not(7)': 12}, 13: {'2': 1, '6': 13}, 14: {'or(0,not(4))': 2, '3': 14}, 15: {'T': 3}}

_TRANS = [(s, g, d) for s in sorted(_DFA.keys()) for g, d in _DFA[s].items()]

_OUT_EDGES = {}
for _t, (_s, _g, _d) in enumerate(_TRANS):
    _OUT_EDGES.setdefault(_s, []).append((_t, _d))
_PATHS = {}
for _e1, (_s, _g, _k) in enumerate(_TRANS):
    for (_e2, _d) in _OUT_EDGES.get(_k, []):
        _PATHS.setdefault((_d, _s), []).append((_e1, _e2))


def _divide_args(guard):
    args = guard.split(',')
    out = []
    i = 0
    while i < len(args):
        a = args[i]
        while a.count('(') != a.count(')'):
            i += 1
            a = a + ',' + args[i]
        out.append(a)
        i += 1
    return out


def _eval_guard(guard, cols):
    if guard[0] == 'a':
        v = 1.0
        for a in _divide_args(guard[4:-1]):
            v = v * _eval_guard(a, cols)
        return v
    elif guard[0] == 'o':
        v = 0.0
        for a in _divide_args(guard[3:-1]):
            e = _eval_guard(a, cols)
            v = v + e - v * e
        return v
    elif guard[0] == 'n':
        return 1.0 - _eval_guard(guard[4:-1], cols)
    elif guard[0] == 'T':
        return jnp.ones_like(cols[0])
    else:
        return cols[int(guard)]


def _build_pair_mats(p_pairs):
    cols_e = [p_pairs[:, k:k + 1] for k in range(_N_SYMBOLS)]
    cols_o = [p_pairs[:, _N_SYMBOLS + k:_N_SYMBOLS + k + 1]
              for k in range(_N_SYMBOLS)]
    ge = [_eval_guard(g, cols_e) for (_, g, _) in _TRANS]
    go = [_eval_guard(g, cols_o) for (_, g, _) in _TRANS]
    zero = jnp.zeros_like(cols_e[0])

    rows = []
    for d in range(_N_STATES):
        row = []
        for s in range(_N_STATES):
            acc = None
            for (e1, e2) in _PATHS.get((d, s), ()):
                term = go[e2] * ge[e1]
                acc = term if acc is None else acc + term
            row.append(zero if acc is None else acc)
        rows.append(jnp.concatenate(row, axis=1)[:, None, :])
    return jnp.concatenate(rows, axis=1)


def _dot(x, y):
    return jax.lax.dot_general(
        x, y, (((1,), (0,)), ((), ())),
        precision=jax.lax.Precision.HIGHEST,
        preferred_element_type=jnp.float32)


def _fuzzy_kernel(p_ref, out_ref, b_ref, c_ref):
    b_ref[:, :, :] = _build_pair_mats(p_ref[:, :])

    for i in range(50):
        c_ref[i] = _dot(b_ref[2 * i + 1], b_ref[2 * i])
    mats = [_dot(c_ref[2 * i + 1], c_ref[2 * i]) for i in range(25)]
    while len(mats) > 1:
        nxt = [_dot(mats[2 * i + 1], mats[2 * i]) for i in range(len(mats) // 2)]
        if len(mats) % 2:
            nxt.append(mats[-1])
        mats = nxt

    out_ref[:, :] = mats[0]


def kernel(symbols_prob):
    p_pairs = symbols_prob.reshape(_SEQ_LEN // 2, 2 * _N_SYMBOLS)
    out = pl.pallas_call(
        _fuzzy_kernel,
        out_shape=jax.ShapeDtypeStruct((_N_STATES, _N_STATES), symbols_prob.dtype),
        scratch_shapes=[
            pltpu.VMEM((_SEQ_LEN // 2, _N_STATES, _N_STATES), symbols_prob.dtype),
            pltpu.VMEM((_SEQ_LEN // 4, _N_STATES, _N_STATES), symbols_prob.dtype),
        ],
    )(p_pairs)
    return out[:, 0]
